# Initial kernel scaffold; baseline (speedup 1.0000x reference)
import functools

import jax
import jax.numpy as jnp
from jax import lax
from jax.experimental import pallas as pl
from jax.experimental.pallas import tpu as pltpu
from jax.experimental.pallas import tpu_sc as plsc

N = 10000
NPAD = 10240
E = 320000
NW = 32
CHUNK = 128
CPW = 79
EPAD = NW * CPW * CHUNK
RPT = NPAD // 16
BM = 1024
GRID = NPAD // BM

_MESH = plsc.VectorSubcoreMesh(core_axis_name="c", subcore_axis_name="s")


def _fill(ref, val, d):
  v = jnp.full((16,), val, jnp.float32)

  def body(i, carry):
    for cblk in range(d // 16):
      ref[i, pl.ds(cblk * 16, 16)] = v
    return carry

  lax.fori_loop(0, CHUNK, body, 0)


def _make_sc_scatter(d):
  def body(g_hbm, src_hbm, dst_hbm, out_hbm, src_v, dst_v, rows_v, acc_sh, sem):
    c = lax.axis_index("c")
    s = lax.axis_index("s")
    wid = s * 2 + c
    pltpu.sync_copy(src_hbm.at[wid], src_v)
    pltpu.sync_copy(dst_hbm.at[wid], dst_v)
    _fill(rows_v, 0.0, d)

    def zacc(j, carry):
      pltpu.sync_copy(rows_v, acc_sh.at[pl.ds(s * RPT + j * CHUNK, CHUNK)])
      return carry

    lax.fori_loop(0, RPT // CHUNK, zacc, 0)
    plsc.subcore_barrier()

    def step(j, carry):
      pltpu.async_copy(g_hbm.at[src_v.at[j]], rows_v, sem).wait()
      pltpu.sync_copy(rows_v, acc_sh.at[dst_v.at[j]], add=True)
      return carry

    lax.fori_loop(0, CPW, step, 0)
    plsc.subcore_barrier()

    def cpout(j, carry):
      off = s * RPT + j * CHUNK
      pltpu.sync_copy(acc_sh.at[pl.ds(off, CHUNK)],
                      out_hbm.at[c, pl.ds(off, CHUNK)])
      return carry

    lax.fori_loop(0, RPT // CHUNK, cpout, 0)

  return pl.kernel(
      body,
      mesh=_MESH,
      out_type=jax.ShapeDtypeStruct((2, NPAD, d), jnp.float32),
      scratch_types=[
          pltpu.VMEM((CPW, CHUNK), jnp.int32),
          pltpu.VMEM((CPW, CHUNK), jnp.int32),
          pltpu.VMEM((CHUNK, d), jnp.float32),
          pltpu.VMEM_SHARED((NPAD, d), jnp.float32),
          pltpu.SemaphoreType.DMA,
      ],
  )


def _make_sc_hist():
  d = 16

  def body(dst_hbm, out_hbm, dst_v, rows_v, acc_sh):
    c = lax.axis_index("c")
    s = lax.axis_index("s")
    wid = s * 2 + c
    pltpu.sync_copy(dst_hbm.at[wid], dst_v)
    _fill(rows_v, 0.0, d)

    def zacc(j, carry):
      pltpu.sync_copy(rows_v, acc_sh.at[pl.ds(s * RPT + j * CHUNK, CHUNK)])
      return carry

    lax.fori_loop(0, RPT // CHUNK, zacc, 0)
    plsc.subcore_barrier()
    _fill(rows_v, 1.0, d)

    def step(j, carry):
      pltpu.sync_copy(rows_v, acc_sh.at[dst_v.at[j]], add=True)
      return carry

    lax.fori_loop(0, CPW, step, 0)
    plsc.subcore_barrier()

    def cpout(j, carry):
      off = s * RPT + j * CHUNK
      pltpu.sync_copy(acc_sh.at[pl.ds(off, CHUNK)],
                      out_hbm.at[c, pl.ds(off, CHUNK)])
      return carry

    lax.fori_loop(0, RPT // CHUNK, cpout, 0)

  return pl.kernel(
      body,
      mesh=_MESH,
      out_type=jax.ShapeDtypeStruct((2, NPAD, d), jnp.float32),
      scratch_types=[
          pltpu.VMEM((CPW, CHUNK), jnp.int32),
          pltpu.VMEM((CHUNK, d), jnp.float32),
          pltpu.VMEM_SHARED((NPAD, d), jnp.float32),
      ],
  )


_SC_HIST = _make_sc_hist()
_SC_SCATTER = {dd: _make_sc_scatter(dd) for dd in (16, 64, 128)}


def _row_spec(d):
  return pl.BlockSpec((BM, d), lambda i: (i, 0))


def _full_spec(r, c):
  return pl.BlockSpec((r, c), lambda i: (0, 0))


def _dinv(d0r, d1r):
  return lax.rsqrt(d0r[:, 0:1] + d1r[:, 0:1] + 1.0)


def _tc_first(d0, d1, x, w):
  din, dout = w.shape

  def f(d0r, d1r, xr, wr, outr):
    dinv = _dinv(d0r, d1r)
    outr[...] = dinv * jnp.dot(xr[...], wr[...],
                               preferred_element_type=jnp.float32)

  return pl.pallas_call(
      f,
      grid=(GRID,),
      in_specs=[_row_spec(16), _row_spec(16), _row_spec(din),
                _full_spec(din, dout)],
      out_specs=_row_spec(dout),
      out_shape=jax.ShapeDtypeStruct((NPAD, dout), jnp.float32),
  )(d0, d1, x, w)


def _tc_mid(d0, d1, a0, a1, g, brow, w):
  din, dout = w.shape

  def f(d0r, d1r, a0r, a1r, gr, br, wr, outr):
    dinv = _dinv(d0r, d1r)
    t = jnp.maximum(dinv * (a0r[...] + a1r[...] + gr[...]) + br[0:1, :], 0.0)
    outr[...] = dinv * jnp.dot(t, wr[...],
                               preferred_element_type=jnp.float32)

  return pl.pallas_call(
      f,
      grid=(GRID,),
      in_specs=[_row_spec(16), _row_spec(16), _row_spec(din), _row_spec(din),
                _row_spec(din), _full_spec(8, din), _full_spec(din, dout)],
      out_specs=_row_spec(dout),
      out_shape=jax.ShapeDtypeStruct((NPAD, dout), jnp.float32),
  )(d0, d1, a0, a1, g, brow, w)


def _tc_final(d0, d1, a0, a1, g, brow):
  d = 16

  def f(d0r, d1r, a0r, a1r, gr, br, outr):
    dinv = _dinv(d0r, d1r)
    outr[...] = jnp.maximum(
        dinv * (a0r[...] + a1r[...] + gr[...]) + br[0:1, :], 0.0)

  return pl.pallas_call(
      f,
      grid=(GRID,),
      in_specs=[_row_spec(16), _row_spec(16), _row_spec(d), _row_spec(d),
                _row_spec(d), _full_spec(8, d)],
      out_specs=_row_spec(d),
      out_shape=jax.ShapeDtypeStruct((NPAD, d), jnp.float32),
  )(d0, d1, a0, a1, g, brow)


def kernel(x, edge_index, W1, b1, W2, b2, W3, b3, W4, b4, W5, b5):
  f32 = jnp.float32
  pad = jnp.full((EPAD - E,), N, jnp.int32)
  src_t = jnp.concatenate([edge_index[0], pad]).reshape(NW, CPW, CHUNK)
  dst_t = jnp.concatenate([edge_index[1], pad]).reshape(NW, CPW, CHUNK)
  xp = jnp.zeros((NPAD, x.shape[1]), f32).at[:N].set(x)

  w4p = jnp.zeros((W4.shape[0], 16), f32).at[:, :2].set(W4)
  w5p = jnp.zeros((16, 16), f32).at[:2, :1].set(W5)

  def brow(b, d):
    return jnp.broadcast_to(
        jnp.zeros((d,), f32).at[:b.shape[0]].set(b), (8, d))

  dacc = _SC_HIST(dst_t)
  d0, d1 = dacc[0], dacc[1]

  g1 = _tc_first(d0, d1, xp, W1)
  a = _SC_SCATTER[64](g1, src_t, dst_t)
  g2 = _tc_mid(d0, d1, a[0], a[1], g1, brow(b1, 64), W2)
  a = _SC_SCATTER[128](g2, src_t, dst_t)
  g3 = _tc_mid(d0, d1, a[0], a[1], g2, brow(b2, 128), W3)
  a = _SC_SCATTER[64](g3, src_t, dst_t)
  g4 = _tc_mid(d0, d1, a[0], a[1], g3, brow(b3, 64), w4p)
  a = _SC_SCATTER[16](g4, src_t, dst_t)
  g5 = _tc_mid(d0, d1, a[0], a[1], g4, brow(b4, 16), w5p)
  a = _SC_SCATTER[16](g5, src_t, dst_t)
  out = _tc_final(d0, d1, a[0], a[1], g5, brow(b5, 16))
  return out[:N, :1]


# speedup vs baseline: 14.4666x; 14.4666x over previous
import functools

import jax
import jax.numpy as jnp
from jax import lax
from jax.experimental import pallas as pl
from jax.experimental.pallas import tpu as pltpu
from jax.experimental.pallas import tpu_sc as plsc

N = 10000
NPAD = 10240
E = 320000
NW = 32
CHUNK = 128
CPW = 79
EPAD = NW * CPW * CHUNK
RPT = NPAD // 16
BM = 1024
GRID = NPAD // BM

_MESH = plsc.VectorSubcoreMesh(core_axis_name="c", subcore_axis_name="s")


def _fill(ref, val, d):
  v = jnp.full((16,), val, jnp.float32)

  def body(i, carry):
    for cblk in range(d // 16):
      ref[i, pl.ds(cblk * 16, 16)] = v
    return carry

  lax.fori_loop(0, CHUNK, body, 0)


def _make_sc_scatter(d):
  def body(g_hbm, src_hbm, dst_hbm, out_hbm, src_v, dst_v, rows_v, acc_sh, sem):
    c = lax.axis_index("c")
    s = lax.axis_index("s")
    wid = s * 2 + c
    pltpu.sync_copy(src_hbm.at[wid], src_v)
    pltpu.sync_copy(dst_hbm.at[wid], dst_v)
    _fill(rows_v, 0.0, d)

    def zacc(j, carry):
      pltpu.sync_copy(rows_v, acc_sh.at[pl.ds(s * RPT + j * CHUNK, CHUNK)])
      return carry

    lax.fori_loop(0, RPT // CHUNK, zacc, 0)
    plsc.subcore_barrier()

    def step(j, carry):
      pltpu.async_copy(g_hbm.at[src_v.at[j]], rows_v, sem).wait()
      pltpu.sync_copy(rows_v, acc_sh.at[dst_v.at[j]], add=True)
      return carry

    lax.fori_loop(0, CPW, step, 0)
    plsc.subcore_barrier()

    def cpout(j, carry):
      off = s * RPT + j * CHUNK
      pltpu.sync_copy(acc_sh.at[pl.ds(off, CHUNK)],
                      out_hbm.at[c, pl.ds(off, CHUNK)])
      return carry

    lax.fori_loop(0, RPT // CHUNK, cpout, 0)

  return pl.kernel(
      body,
      mesh=_MESH,
      out_type=jax.ShapeDtypeStruct((2, NPAD, d), jnp.float32),
      compiler_params=pltpu.CompilerParams(use_tc_tiling_on_sc=False),
      scratch_types=[
          pltpu.VMEM((CPW, CHUNK), jnp.int32),
          pltpu.VMEM((CPW, CHUNK), jnp.int32),
          pltpu.VMEM((CHUNK, d), jnp.float32),
          pltpu.VMEM_SHARED((NPAD, d), jnp.float32),
          pltpu.SemaphoreType.DMA,
      ],
  )


def _make_sc_hist():
  d = 16

  def body(dst_hbm, out_hbm, dst_v, rows_v, acc_sh):
    c = lax.axis_index("c")
    s = lax.axis_index("s")
    wid = s * 2 + c
    pltpu.sync_copy(dst_hbm.at[wid], dst_v)
    _fill(rows_v, 0.0, d)

    def zacc(j, carry):
      pltpu.sync_copy(rows_v, acc_sh.at[pl.ds(s * RPT + j * CHUNK, CHUNK)])
      return carry

    lax.fori_loop(0, RPT // CHUNK, zacc, 0)
    plsc.subcore_barrier()
    _fill(rows_v, 1.0, d)

    def step(j, carry):
      pltpu.sync_copy(rows_v, acc_sh.at[dst_v.at[j]], add=True)
      return carry

    lax.fori_loop(0, CPW, step, 0)
    plsc.subcore_barrier()

    def cpout(j, carry):
      off = s * RPT + j * CHUNK
      pltpu.sync_copy(acc_sh.at[pl.ds(off, CHUNK)],
                      out_hbm.at[c, pl.ds(off, CHUNK)])
      return carry

    lax.fori_loop(0, RPT // CHUNK, cpout, 0)

  return pl.kernel(
      body,
      mesh=_MESH,
      out_type=jax.ShapeDtypeStruct((2, NPAD, d), jnp.float32),
      compiler_params=pltpu.CompilerParams(use_tc_tiling_on_sc=False),
      scratch_types=[
          pltpu.VMEM((CPW, CHUNK), jnp.int32),
          pltpu.VMEM((CHUNK, d), jnp.float32),
          pltpu.VMEM_SHARED((NPAD, d), jnp.float32),
      ],
  )


_SC_HIST = _make_sc_hist()
_SC_SCATTER = {dd: _make_sc_scatter(dd) for dd in (16, 64, 128)}


def _row_spec(d):
  return pl.BlockSpec((BM, d), lambda i: (i, 0))


def _full_spec(r, c):
  return pl.BlockSpec((r, c), lambda i: (0, 0))


def _dinv(d0r, d1r):
  return lax.rsqrt(d0r[:, 0:1] + d1r[:, 0:1] + 1.0)


def _tc_first(d0, d1, x, w):
  din, dout = w.shape

  def f(d0r, d1r, xr, wr, outr):
    dinv = _dinv(d0r, d1r)
    outr[...] = dinv * jnp.dot(xr[...], wr[...],
                               preferred_element_type=jnp.float32)

  return pl.pallas_call(
      f,
      grid=(GRID,),
      in_specs=[_row_spec(16), _row_spec(16), _row_spec(din),
                _full_spec(din, dout)],
      out_specs=_row_spec(dout),
      out_shape=jax.ShapeDtypeStruct((NPAD, dout), jnp.float32),
  )(d0, d1, x, w)


def _tc_mid(d0, d1, a0, a1, g, brow, w):
  din, dout = w.shape

  def f(d0r, d1r, a0r, a1r, gr, br, wr, outr):
    dinv = _dinv(d0r, d1r)
    t = jnp.maximum(dinv * (a0r[...] + a1r[...] + gr[...]) + br[0:1, :], 0.0)
    outr[...] = dinv * jnp.dot(t, wr[...],
                               preferred_element_type=jnp.float32)

  return pl.pallas_call(
      f,
      grid=(GRID,),
      in_specs=[_row_spec(16), _row_spec(16), _row_spec(din), _row_spec(din),
                _row_spec(din), _full_spec(8, din), _full_spec(din, dout)],
      out_specs=_row_spec(dout),
      out_shape=jax.ShapeDtypeStruct((NPAD, dout), jnp.float32),
  )(d0, d1, a0, a1, g, brow, w)


def _tc_final(d0, d1, a0, a1, g, brow):
  d = 16

  def f(d0r, d1r, a0r, a1r, gr, br, outr):
    dinv = _dinv(d0r, d1r)
    outr[...] = jnp.maximum(
        dinv * (a0r[...] + a1r[...] + gr[...]) + br[0:1, :], 0.0)

  return pl.pallas_call(
      f,
      grid=(GRID,),
      in_specs=[_row_spec(16), _row_spec(16), _row_spec(d), _row_spec(d),
                _row_spec(d), _full_spec(8, d)],
      out_specs=_row_spec(d),
      out_shape=jax.ShapeDtypeStruct((NPAD, d), jnp.float32),
  )(d0, d1, a0, a1, g, brow)


def kernel(x, edge_index, W1, b1, W2, b2, W3, b3, W4, b4, W5, b5):
  f32 = jnp.float32
  pad = jnp.full((EPAD - E,), N, jnp.int32)
  src_t = jnp.concatenate([edge_index[0], pad]).reshape(NW, CPW, CHUNK)
  dst_t = jnp.concatenate([edge_index[1], pad]).reshape(NW, CPW, CHUNK)
  xp = jnp.zeros((NPAD, x.shape[1]), f32).at[:N].set(x)

  w4p = jnp.zeros((W4.shape[0], 16), f32).at[:, :2].set(W4)
  w5p = jnp.zeros((16, 16), f32).at[:2, :1].set(W5)

  def brow(b, d):
    return jnp.broadcast_to(
        jnp.zeros((d,), f32).at[:b.shape[0]].set(b), (8, d))

  dacc = _SC_HIST(dst_t)
  d0, d1 = dacc[0], dacc[1]

  g1 = _tc_first(d0, d1, xp, W1)
  a = _SC_SCATTER[64](g1, src_t, dst_t)
  g2 = _tc_mid(d0, d1, a[0], a[1], g1, brow(b1, 64), W2)
  a = _SC_SCATTER[128](g2, src_t, dst_t)
  g3 = _tc_mid(d0, d1, a[0], a[1], g2, brow(b2, 128), W3)
  a = _SC_SCATTER[64](g3, src_t, dst_t)
  g4 = _tc_mid(d0, d1, a[0], a[1], g3, brow(b3, 64), w4p)
  a = _SC_SCATTER[16](g4, src_t, dst_t)
  g5 = _tc_mid(d0, d1, a[0], a[1], g4, brow(b4, 16), w5p)
  a = _SC_SCATTER[16](g5, src_t, dst_t)
  out = _tc_final(d0, d1, a[0], a[1], g5, brow(b5, 16))
  return out[:N, :1]
